# baseline (speedup 1.0000x reference)
import jax
import jax.numpy as jnp
from jax import lax
from jax.experimental import pallas as pl
from jax.experimental.pallas import tpu as pltpu
from jax.experimental.pallas import tpu_sc as plsc

VOCAB = 1000000
BATCH = 16384
FIELDS = 26
EMBED = 64
NUM_WORKERS = 32
NBLK = BATCH // 128
BLK_PER_W = NBLK // NUM_WORKERS
ITEMS = FIELDS * BLK_PER_W
GPAD = 129


def _body(idx_hbm, table_hbm, out_hbm, idx_v, gbufs, obufs, gsem, ssem):
    wid = lax.axis_index("s") * 2 + lax.axis_index("c")
    w4 = wid * BLK_PER_W
    pltpu.sync_copy(idx_hbm.at[:, pl.ds(w4, BLK_PER_W), :], idx_v)

    iota = lax.iota(jnp.int32, 16)
    e8s = [(iota + g * 16) // 8 for g in range(4)]
    ers = [lax.rem(iota + g * 16, 8) for g in range(4)]

    def fire_gather(k, b):
        f = k // BLK_PER_W
        j = lax.rem(k, BLK_PER_W)
        pltpu.async_copy(table_hbm.at[idx_v.at[f, j]], gbufs[b], gsem)

    def wait_gather(b):
        pltpu.make_async_copy(
            table_hbm.at[idx_v.at[0, 0]], gbufs[b], gsem).wait()

    def fire_store(k, b):
        f = k // BLK_PER_W
        blk = w4 + lax.rem(k, BLK_PER_W)
        pltpu.async_copy(
            obufs[b].at[:, :, pl.ds(0, 128)], out_hbm.at[f, :, blk], ssem)

    def wait_store(b):
        pltpu.make_async_copy(
            obufs[b].at[:, :, pl.ds(0, 128)], out_hbm.at[0, :, 0], ssem).wait()

    def select(b):
        def inner(c4, carry):
            c0 = c4 * 4
            vecs = []
            for q in range(4):
                for g in range(4):
                    vecs.append(gbufs[b][c0 + q, pl.ds(g * 16, 16)])
            for q in range(4):
                cs = jnp.full((16,), 0, jnp.int32) + (c0 + q)
                for g in range(4):
                    plsc.store_scatter(
                        obufs[b], [e8s[g], ers[g], cs], vecs[q * 4 + g])
            return carry

        lax.fori_loop(0, 32, inner, 0)

    fire_gather(0, 0)

    def outer(k2, carry):
        k = k2 * 2
        for p in range(2):
            kk = k + p

            @pl.when(kk + 1 < ITEMS)
            def _():
                fire_gather(kk + 1, 1 - p)

            wait_gather(p)

            @pl.when(kk >= 2)
            def _():
                wait_store(p)

            select(p)
            fire_store(kk, p)
        return carry

    lax.fori_loop(0, ITEMS // 2, outer, 0)
    wait_store(0)
    wait_store(1)


def kernel(indices, E):
    idx3 = jnp.transpose(indices).reshape(FIELDS, NBLK, 128).astype(jnp.int32)
    mesh = plsc.VectorSubcoreMesh(core_axis_name="c", subcore_axis_name="s")
    run = pl.kernel(
        _body,
        out_type=jax.ShapeDtypeStruct((FIELDS, 8, NBLK, 8, 128), jnp.float32),
        mesh=mesh,
        scratch_types=[
            pltpu.VMEM((FIELDS, BLK_PER_W, 128), jnp.int32),
            [pltpu.VMEM((128, 128), jnp.float32) for _ in range(2)],
            [pltpu.VMEM((8, 8, GPAD), jnp.float32) for _ in range(2)],
            pltpu.SemaphoreType.DMA,
            pltpu.SemaphoreType.DMA,
        ],
        compiler_params=pltpu.CompilerParams(
            use_tc_tiling_on_sc=True, needs_layout_passes=False),
    )
    p_out = run(idx3, jnp.pad(E, ((0, 0), (0, EMBED))))
    return p_out.transpose(2, 4, 0, 1, 3).reshape(BATCH, FIELDS, EMBED)


# speedup vs baseline: 1.3522x; 1.3522x over previous
import jax
import jax.numpy as jnp
from jax import lax
from jax.experimental import pallas as pl
from jax.experimental.pallas import tpu as pltpu
from jax.experimental.pallas import tpu_sc as plsc

VOCAB = 1000000
BATCH = 16384
FIELDS = 26
EMBED = 64
NUM_WORKERS = 32
NBLK = BATCH // 128
BLK_PER_W = NBLK // NUM_WORKERS
ITEMS = FIELDS * BLK_PER_W
GPAD = 129


def _body(idx_hbm, table_hbm, out_hbm, idx_v, gbufs, obufs, gsem, ssem):
    wid = lax.axis_index("s") * 2 + lax.axis_index("c")
    w4 = wid * BLK_PER_W
    pltpu.sync_copy(idx_hbm.at[:, pl.ds(w4, BLK_PER_W), :], idx_v)

    iota = lax.iota(jnp.int32, 16)
    e8s = [(iota + g * 16) // 8 for g in range(4)]
    ers = [lax.rem(iota + g * 16, 8) for g in range(4)]

    def fire_gather(k, b):
        f = k // BLK_PER_W
        j = lax.rem(k, BLK_PER_W)
        pltpu.async_copy(table_hbm.at[idx_v.at[f, j]], gbufs[b], gsem)

    def wait_gather(b):
        pltpu.make_async_copy(
            table_hbm.at[idx_v.at[0, 0]], gbufs[b], gsem).wait()

    def fire_store(k, b):
        f = k // BLK_PER_W
        blk = w4 + lax.rem(k, BLK_PER_W)
        pltpu.async_copy(
            obufs[b].at[:, :, pl.ds(0, 128)], out_hbm.at[f, :, blk], ssem)

    def wait_store(b):
        pltpu.make_async_copy(
            obufs[b].at[:, :, pl.ds(0, 128)], out_hbm.at[0, :, 0], ssem).wait()

    def select(b):
        def inner(c4, carry):
            c0 = c4 * 4
            vecs = []
            for q in range(4):
                for g in range(4):
                    vecs.append(gbufs[b][c0 + q, pl.ds(g * 16, 16)])
            for q in range(4):
                cs = jnp.full((16,), 0, jnp.int32) + (c0 + q)
                for g in range(4):
                    plsc.store_scatter(
                        obufs[b], [e8s[g], ers[g], cs], vecs[q * 4 + g])
            return carry

        lax.fori_loop(0, 32, inner, 0)

    for b in range(3):
        fire_gather(b, b)

    def outer(k4, carry):
        k = k4 * 4
        for p in range(4):
            kk = k + p
            q = (p + 3) % 4

            @pl.when(kk >= 1)
            def _():
                wait_store(q)

            @pl.when(kk + 3 < ITEMS)
            def _():
                fire_gather(kk + 3, q)

            wait_gather(p)
            select(p)
            fire_store(kk, p)
        return carry

    lax.fori_loop(0, ITEMS // 4, outer, 0)
    wait_store(3)


def kernel(indices, E):
    idx3 = jnp.transpose(indices).reshape(FIELDS, NBLK, 128).astype(jnp.int32)
    mesh = plsc.VectorSubcoreMesh(core_axis_name="c", subcore_axis_name="s")
    run = pl.kernel(
        _body,
        out_type=jax.ShapeDtypeStruct((FIELDS, 8, NBLK, 8, 128), jnp.float32),
        mesh=mesh,
        scratch_types=[
            pltpu.VMEM((FIELDS, BLK_PER_W, 128), jnp.int32),
            [pltpu.VMEM((128, EMBED), jnp.float32) for _ in range(4)],
            [pltpu.VMEM((8, 8, GPAD), jnp.float32) for _ in range(4)],
            pltpu.SemaphoreType.DMA,
            pltpu.SemaphoreType.DMA,
        ],
        compiler_params=pltpu.CompilerParams(
            use_tc_tiling_on_sc=False, needs_layout_passes=False),
    )
    p_out = run(idx3, E)
    return p_out.transpose(2, 4, 0, 1, 3).reshape(BATCH, FIELDS, EMBED)


# speedup vs baseline: 1.3927x; 1.0299x over previous
import jax
import jax.numpy as jnp
from jax import lax
from jax.experimental import pallas as pl
from jax.experimental.pallas import tpu as pltpu
from jax.experimental.pallas import tpu_sc as plsc

VOCAB = 1000000
BATCH = 16384
FIELDS = 26
EMBED = 64
NUM_WORKERS = 32
NBLK = BATCH // 128
BLK_PER_W = NBLK // NUM_WORKERS
ITEMS = FIELDS * BLK_PER_W
GPAD = 129


def _body(idx_hbm, table_hbm, out_hbm, idx_v, gbufs, obufs, gsem, ssem):
    wid = lax.axis_index("s") * 2 + lax.axis_index("c")
    w4 = wid * BLK_PER_W
    pltpu.sync_copy(idx_hbm.at[:, pl.ds(w4, BLK_PER_W), :], idx_v)

    iota = lax.iota(jnp.int32, 16)
    e8s = [(iota + g * 16) // 8 for g in range(4)]
    ers = [lax.rem(iota + g * 16, 8) for g in range(4)]

    def fire_gather(k, b):
        f = k // BLK_PER_W
        j = lax.rem(k, BLK_PER_W)
        pltpu.async_copy(table_hbm.at[idx_v.at[f, j]], gbufs[b], gsem)

    def wait_gather(b):
        pltpu.make_async_copy(
            table_hbm.at[idx_v.at[0, 0]], gbufs[b], gsem).wait()

    def fire_store(k, b):
        f = k // BLK_PER_W
        blk = w4 + lax.rem(k, BLK_PER_W)
        pltpu.async_copy(
            obufs[b].at[:, :, pl.ds(0, 128)], out_hbm.at[f, :, blk], ssem)

    def wait_store(b):
        pltpu.make_async_copy(
            obufs[b].at[:, :, pl.ds(0, 128)], out_hbm.at[0, :, 0], ssem).wait()

    def select(b):
        def inner(c4, carry):
            c0 = c4 * 4
            vecs = []
            for q in range(4):
                for g in range(4):
                    vecs.append(gbufs[b][c0 + q, pl.ds(g * 16, 16)])
            for q in range(4):
                cs = jnp.full((16,), 0, jnp.int32) + (c0 + q)
                for g in range(4):
                    plsc.store_scatter(
                        obufs[b], [e8s[g], ers[g], cs], vecs[q * 4 + g])
            return carry

        lax.fori_loop(0, 32, inner, 0)

    fire_gather(0, 0)

    def outer(k2, carry):
        k = k2 * 2
        for p in range(2):
            kk = k + p

            @pl.when(kk + 1 < ITEMS)
            def _():
                fire_gather(kk + 1, 1 - p)

            wait_gather(p)

            @pl.when(kk >= 2)
            def _():
                wait_store(p)

            select(p)
            fire_store(kk, p)
        return carry

    lax.fori_loop(0, ITEMS // 2, outer, 0)
    wait_store(0)
    wait_store(1)


def kernel(indices, E):
    idx3 = jnp.transpose(indices).reshape(FIELDS, NBLK, 128).astype(jnp.int32)
    mesh = plsc.VectorSubcoreMesh(core_axis_name="c", subcore_axis_name="s")
    run = pl.kernel(
        _body,
        out_type=jax.ShapeDtypeStruct((FIELDS, 8, NBLK, 8, 128), jnp.float32),
        mesh=mesh,
        scratch_types=[
            pltpu.VMEM((FIELDS, BLK_PER_W, 128), jnp.int32),
            [pltpu.VMEM((128, EMBED), jnp.float32) for _ in range(2)],
            [pltpu.VMEM((8, 8, GPAD), jnp.float32) for _ in range(2)],
            pltpu.SemaphoreType.DMA,
            pltpu.SemaphoreType.DMA,
        ],
        compiler_params=pltpu.CompilerParams(
            use_tc_tiling_on_sc=False, needs_layout_passes=False),
    )
    p_out = run(idx3, E)
    return p_out.transpose(2, 4, 0, 1, 3).reshape(BATCH, FIELDS, EMBED)
